# baseline (speedup 1.0000x reference)
import jax
import jax.numpy as jnp
from jax import lax
from jax.experimental import pallas as pl
from jax.experimental.pallas import tpu as pltpu

_B, _A, _D = 100, 100, 128
_K = 32
_CH = 25
_G = 10


def _gnn_kernel(obs_ref, w1_ref, b1_ref, w2_ref, b2_ref, wout_ref, bout_ref,
                out_ref):
    x3 = obs_ref[...]

    px = x3[:, :, 0:1]
    py = x3[:, :, 1:2]
    pxt = jnp.transpose(px, (0, 2, 1))
    pyt = jnp.transpose(py, (0, 2, 1))
    dx = px - pxt
    dy = py - pyt
    dist2 = dx * dx + dy * dy

    col = lax.broadcasted_iota(jnp.int32, (_G, _A, _A), 2)
    row = lax.broadcasted_iota(jnp.int32, (_G, _A, _A), 1)
    big = jnp.float32(jnp.inf)

    def body(_, d):
        m = jnp.min(d, axis=2, keepdims=True)
        eq = d == m
        mi = jnp.min(jnp.where(eq, col, _A), axis=2, keepdims=True)
        return jnp.where(col == mi, big, d)

    dfin = lax.fori_loop(0, _K, body, dist2)
    nbr = jnp.where((dfin == big) & (row != col),
                    jnp.float32(1.0), jnp.float32(0.0))
    nbrT = jnp.transpose(nbr, (0, 2, 1))

    def layer(xin, w_ref, b_ref):
        xw = jnp.dot(xin, w_ref[...], preferred_element_type=jnp.float32)
        parts = []
        for j0 in range(0, _A, _CH):
            acc = jnp.zeros((_G, _CH, _D), jnp.float32)
            for a in range(_A):
                acc = acc + nbrT[:, j0:j0 + _CH, a:a + 1] * xw[:, a:a + 1, :]
            parts.append(acc)
        agg = jnp.concatenate(parts, axis=1)
        return jnp.tanh(agg + xw + b_ref[...][None])

    h = layer(x3, w1_ref, b1_ref)
    h = layer(h, w2_ref, b2_ref)
    vals = jnp.dot(h, wout_ref[...], preferred_element_type=jnp.float32)
    out_ref[...] = vals + bout_ref[...][None]


def kernel(agent_observations, W1, b1, W2, b2, Wout, bout):
    b1r = b1.reshape(1, -1)
    b2r = b2.reshape(1, -1)
    boutr = bout.reshape(1, 1)
    out = pl.pallas_call(
        _gnn_kernel,
        grid=(_B // _G,),
        in_specs=[
            pl.BlockSpec((_G, _A, _D), lambda b: (b, 0, 0)),
            pl.BlockSpec((_D, _D), lambda b: (0, 0)),
            pl.BlockSpec((1, _D), lambda b: (0, 0)),
            pl.BlockSpec((_D, _D), lambda b: (0, 0)),
            pl.BlockSpec((1, _D), lambda b: (0, 0)),
            pl.BlockSpec((_D, 1), lambda b: (0, 0)),
            pl.BlockSpec((1, 1), lambda b: (0, 0)),
        ],
        out_specs=pl.BlockSpec((_G, _A, 1), lambda b: (b, 0, 0)),
        out_shape=jax.ShapeDtypeStruct((_B, _A, 1), jnp.float32),
        compiler_params=pltpu.CompilerParams(
            dimension_semantics=("parallel",)),
    )(agent_observations, W1, b1r, W2, b2r, Wout, boutr)
    return out


# speedup vs baseline: 1.0055x; 1.0055x over previous
import jax
import jax.numpy as jnp
from jax import lax
from jax.experimental import pallas as pl
from jax.experimental.pallas import tpu as pltpu

_B, _A, _D = 100, 100, 128
_K = 32
_CH = 50
_G = 10


def _gnn_kernel(obs_ref, w1_ref, b1_ref, w2_ref, b2_ref, wout_ref, bout_ref,
                out_ref):
    x3 = obs_ref[...]

    px = x3[:, :, 0:1]
    py = x3[:, :, 1:2]
    pxt = jnp.transpose(px, (0, 2, 1))
    pyt = jnp.transpose(py, (0, 2, 1))
    dx = px - pxt
    dy = py - pyt
    dist2 = dx * dx + dy * dy

    col = lax.broadcasted_iota(jnp.int32, (_G, _A, _A), 2)
    row = lax.broadcasted_iota(jnp.int32, (_G, _A, _A), 1)
    big = jnp.float32(jnp.inf)

    def body(_, d):
        m = jnp.min(d, axis=2, keepdims=True)
        eq = d == m
        mi = jnp.min(jnp.where(eq, col, _A), axis=2, keepdims=True)
        return jnp.where(col == mi, big, d)

    dfin = lax.fori_loop(0, _K, body, dist2)
    nbr = jnp.where((dfin == big) & (row != col),
                    jnp.float32(1.0), jnp.float32(0.0))
    nbrT = jnp.transpose(nbr, (0, 2, 1))

    def layer(xin, w_ref, b_ref):
        xw = jnp.dot(xin, w_ref[...], preferred_element_type=jnp.float32)
        parts = []
        for j0 in range(0, _A, _CH):
            acc = jnp.zeros((_G, _CH, _D), jnp.float32)
            for a in range(_A):
                acc = acc + nbrT[:, j0:j0 + _CH, a:a + 1] * xw[:, a:a + 1, :]
            parts.append(acc)
        agg = jnp.concatenate(parts, axis=1)
        return jnp.tanh(agg + xw + b_ref[...][None])

    h = layer(x3, w1_ref, b1_ref)
    h = layer(h, w2_ref, b2_ref)
    vals = jnp.dot(h, wout_ref[...], preferred_element_type=jnp.float32)
    out_ref[...] = vals + bout_ref[...][None]


def kernel(agent_observations, W1, b1, W2, b2, Wout, bout):
    b1r = b1.reshape(1, -1)
    b2r = b2.reshape(1, -1)
    boutr = bout.reshape(1, 1)
    out = pl.pallas_call(
        _gnn_kernel,
        grid=(_B // _G,),
        in_specs=[
            pl.BlockSpec((_G, _A, _D), lambda b: (b, 0, 0)),
            pl.BlockSpec((_D, _D), lambda b: (0, 0)),
            pl.BlockSpec((1, _D), lambda b: (0, 0)),
            pl.BlockSpec((_D, _D), lambda b: (0, 0)),
            pl.BlockSpec((1, _D), lambda b: (0, 0)),
            pl.BlockSpec((_D, 1), lambda b: (0, 0)),
            pl.BlockSpec((1, 1), lambda b: (0, 0)),
        ],
        out_specs=pl.BlockSpec((_G, _A, 1), lambda b: (b, 0, 0)),
        out_shape=jax.ShapeDtypeStruct((_B, _A, 1), jnp.float32),
        compiler_params=pltpu.CompilerParams(
            dimension_semantics=("parallel",)),
    )(agent_observations, W1, b1r, W2, b2r, Wout, boutr)
    return out
